# baseline (speedup 1.0000x reference)
import functools

import jax
import jax.numpy as jnp
from jax import lax
from jax.experimental import pallas as pl
from jax.experimental.pallas import tpu as pltpu
from jax.experimental.pallas import tpu_sc as plsc

D_MODEL = 64
DP = 128
NUM_CORES = 2
NUM_SUBCORES = 16
NW = NUM_CORES * NUM_SUBCORES
CHUNK = 128
NB = 5


@functools.cache
def _make_kernel(total: int):
    per_w = total // NW
    n_chunks = per_w // CHUNK
    mesh = plsc.VectorSubcoreMesh(core_axis_name="c", subcore_axis_name="s")

    @functools.partial(
        pl.kernel,
        mesh=mesh,
        out_type=jax.ShapeDtypeStruct((total, DP), jnp.float32),
        scratch_types=[
            pltpu.VMEM((per_w,), jnp.int32),
            pltpu.VMEM((NB, CHUNK, DP), jnp.float32),
            pltpu.SemaphoreType.DMA((NB,)),
            pltpu.SemaphoreType.DMA((NB,)),
        ],
    )
    def emb_kernel(idx_hbm, table_hbm, out_hbm, idx_v, rows, gsem, wsem):
        wid = lax.axis_index("s") * NUM_CORES + lax.axis_index("c")
        base = wid * per_w
        pltpu.sync_copy(idx_hbm.at[pl.ds(base, per_w)], idx_v)

        def gather(j, b):
            return pltpu.make_async_copy(
                table_hbm.at[idx_v.at[pl.ds(j * CHUNK, CHUNK)]],
                rows.at[b], gsem.at[b])

        def write(j, b):
            return pltpu.make_async_copy(
                rows.at[b],
                out_hbm.at[pl.ds(base + j * CHUNK, CHUNK)],
                wsem.at[b])

        LG = 3
        WD = NB - LG

        for j in range(LG):
            gather(j, j).start()

        def outer(j0, _):
            for k in range(NB):
                j = j0 * NB + k
                gather(j, k).wait()
                write(j, k).start()

                @pl.when(j >= WD)
                def _():
                    write(j - WD, (k - WD) % NB).wait()

                @pl.when(j + LG < n_chunks)
                def _():
                    gather(j + LG, (k + LG) % NB).start()
            return ()

        lax.fori_loop(0, n_chunks // NB, outer, (), unroll=False)

        for j in range(n_chunks - WD, n_chunks):
            write(j, j % NB).wait()

    return emb_kernel


def kernel(inputs, table):
    batch, hist = inputs.shape
    total = batch * hist
    assert total % (NW * CHUNK) == 0
    idx = inputs.astype(jnp.int32).reshape(total)
    table_p = jnp.pad(table, ((0, 0), (0, DP - table.shape[1])))
    out = _make_kernel(total)(idx, table_p)
    return out[:, :D_MODEL].reshape(batch, hist, table.shape[1])
